# Initial kernel scaffold; baseline (speedup 1.0000x reference)
import jax
import jax.numpy as jnp
from jax.experimental import pallas as pl


def kernel(node_type, edge_type, edge_index, edge_label, node_emb, edge_emb, W1, b1, W2, b2, W3, b3, Wh1, bh1, Wh2, bh2):
    raise NotImplementedError("write your pallas kernel here")


# speedup vs baseline: 8.3346x; 8.3346x over previous
import functools

import jax
import jax.numpy as jnp
from jax import lax
from jax.experimental import pallas as pl
from jax.experimental.pallas import tpu as pltpu
from jax.experimental.pallas import tpu_sc as plsc

N = 10000
NP = 10240
HID = 128
E = 320000
CH = 128
NCH = 79
EPT = CH * NCH
EPAD = EPT * 32
TRASH = 10200
RPS = NP // 16
BLK = 1024
F32 = jnp.float32
HIGH = lax.Precision.HIGHEST

@functools.cache
def _mesh():
    return plsc.VectorSubcoreMesh(core_axis_name="c", subcore_axis_name="s",
                                  num_cores=2, num_subcores=16)


def _deg_body(dst_hbm, out_hbm, didx, ones, zb, dsh):
    cid = lax.axis_index("c")
    sid = lax.axis_index("s")
    wid = cid * 16 + sid

    def init_ones(i, c):
        ones[pl.ds(i * 16, 16)] = jnp.ones((16,), F32)
        return c

    lax.fori_loop(0, CH // 16, init_ones, 0)

    def init_zb(i, c):
        zb[pl.ds(i * 16, 16)] = jnp.zeros((16,), F32)
        return c

    lax.fori_loop(0, RPS // 16, init_zb, 0)
    pltpu.sync_copy(zb, dsh.at[pl.ds(sid * RPS, RPS)])
    plsc.subcore_barrier()

    def chunk(g, c):
        off = wid * EPT + g * CH
        pltpu.sync_copy(dst_hbm.at[pl.ds(off, CH)], didx)
        pltpu.sync_copy(ones, dsh.at[didx], add=True)
        return c

    lax.fori_loop(0, NCH, chunk, 0)
    plsc.subcore_barrier()
    pltpu.sync_copy(dsh.at[pl.ds(sid * RPS, RPS)],
                    out_hbm.at[pl.ds(cid * NP + sid * RPS, RPS)])


@functools.cache
def _deg_kernel():
    return pl.kernel(
        _deg_body,
        out_type=jax.ShapeDtypeStruct((2 * NP,), F32),
        mesh=_mesh(),
        scratch_types=[
            pltpu.VMEM((CH,), jnp.int32),
            pltpu.VMEM((CH,), F32),
            pltpu.VMEM((RPS,), F32),
            pltpu.VMEM_SHARED((NP,), F32),
        ],
    )


def _sc_degree(dstp):
    return _deg_kernel()(dstp)


def _scat_body(y_hbm, src_hbm, dst_hbm, zero_hbm, out_hbm,
               sidx, didx, rows, zsh, sem):
    cid = lax.axis_index("c")
    sid = lax.axis_index("s")
    wid = cid * 16 + sid
    pltpu.sync_copy(zero_hbm.at[pl.ds(sid * RPS, RPS)],
                    zsh.at[pl.ds(sid * RPS, RPS)])
    plsc.subcore_barrier()

    def chunk(g, c):
        off = wid * EPT + g * CH
        pltpu.sync_copy(src_hbm.at[pl.ds(off, CH)], sidx)
        pltpu.sync_copy(dst_hbm.at[pl.ds(off, CH)], didx)
        pltpu.async_copy(y_hbm.at[sidx], rows, sem).wait()
        pltpu.sync_copy(rows, zsh.at[didx], add=True)
        return c

    lax.fori_loop(0, NCH, chunk, 0)
    plsc.subcore_barrier()
    pltpu.sync_copy(zsh.at[pl.ds(sid * RPS, RPS)],
                    out_hbm.at[pl.ds(cid * NP + sid * RPS, RPS)])


@functools.cache
def _scat_kernel():
    return pl.kernel(
        _scat_body,
        out_type=jax.ShapeDtypeStruct((2 * NP, HID), F32),
        mesh=_mesh(),
        scratch_types=[
            pltpu.VMEM((CH,), jnp.int32),
            pltpu.VMEM((CH,), jnp.int32),
            pltpu.VMEM((CH, HID), F32),
            pltpu.VMEM_SHARED((NP, HID), F32),
            pltpu.SemaphoreType.DMA,
        ],
    )


def _sc_scatter(y, srcp, dstp, zeros2d):
    return _scat_kernel()(y, srcp, dstp, zeros2d)


def _prep_body(deg_ref, nt_ref, emb_ref, w_ref, y_ref, dinv_ref):
    deg = deg_ref[0, :] + deg_ref[1, :] + 1.0
    dinv = lax.rsqrt(deg)
    table = jnp.dot(emb_ref[...], w_ref[...], precision=HIGH)
    nt = nt_ref[...]
    oh = (nt[:, None] == lax.broadcasted_iota(jnp.int32, (BLK, 4), 1)).astype(F32)
    x = jnp.dot(oh, table, precision=HIGH)
    y_ref[...] = x * dinv[:, None]
    dinv_ref[...] = dinv


def _tc_prep(deg2, ntp, node_emb, w1):
    return pl.pallas_call(
        _prep_body,
        grid=(NP // BLK,),
        in_specs=[
            pl.BlockSpec((2, BLK), lambda i: (0, i)),
            pl.BlockSpec((BLK,), lambda i: (i,)),
            pl.BlockSpec((4, HID), lambda i: (0, 0)),
            pl.BlockSpec((HID, HID), lambda i: (0, 0)),
        ],
        out_specs=[
            pl.BlockSpec((BLK, HID), lambda i: (i, 0)),
            pl.BlockSpec((BLK,), lambda i: (i,)),
        ],
        out_shape=[
            jax.ShapeDtypeStruct((NP, HID), F32),
            jax.ShapeDtypeStruct((NP,), F32),
        ],
    )(deg2, ntp, node_emb, w1)


def _mid_body(z0_ref, z1_ref, y_ref, dinv_ref, b_ref, w_ref, yn_ref):
    dv = dinv_ref[...]
    x = jnp.maximum(
        dv[:, None] * (z0_ref[...] + z1_ref[...] + y_ref[...])
        + b_ref[...][None, :], 0.0)
    yn_ref[...] = jnp.dot(x, w_ref[...], precision=HIGH) * dv[:, None]


def _tc_mid(z0, z1, y, dinv, b, w_next):
    return pl.pallas_call(
        _mid_body,
        grid=(NP // BLK,),
        in_specs=[
            pl.BlockSpec((BLK, HID), lambda i: (i, 0)),
            pl.BlockSpec((BLK, HID), lambda i: (i, 0)),
            pl.BlockSpec((BLK, HID), lambda i: (i, 0)),
            pl.BlockSpec((BLK,), lambda i: (i,)),
            pl.BlockSpec((HID,), lambda i: (0,)),
            pl.BlockSpec((HID, HID), lambda i: (0, 0)),
        ],
        out_specs=pl.BlockSpec((BLK, HID), lambda i: (i, 0)),
        out_shape=jax.ShapeDtypeStruct((NP, HID), F32),
    )(z0, z1, y, dinv, b, w_next)


def _head_body(z0s_ref, z1s_ref, ys_ref, dvs_ref,
               z0d_ref, z1d_ref, yd_ref, dvd_ref,
               b3_ref, wh1_ref, bh1_ref, wh2_ref, bh2_ref, out_ref):
    b3 = b3_ref[...][None, :]
    xs = jnp.maximum(
        dvs_ref[...][:, None] * (z0s_ref[...] + z1s_ref[...] + ys_ref[...]) + b3,
        0.0)
    xd = jnp.maximum(
        dvd_ref[...][:, None] * (z0d_ref[...] + z1d_ref[...] + yd_ref[...]) + b3,
        0.0)
    h = jnp.maximum(
        jnp.dot(xs, wh1_ref[0:HID, :], precision=HIGH)
        + jnp.dot(xd, wh1_ref[HID:2 * HID, :], precision=HIGH)
        + bh1_ref[...][None, :], 0.0)
    out_ref[...] = jnp.dot(h, wh2_ref[...], precision=HIGH) + bh2_ref[...][None, :]


def _tc_head(bs, z0, z1, y, dinv, b3, wh1, bh1, wh2, bh2):
    return pl.pallas_call(
        _head_body,
        out_shape=jax.ShapeDtypeStruct((bs, 1), F32),
    )(z0[:bs], z1[:bs], y[:bs], dinv[:bs],
      z0[bs:2 * bs], z1[bs:2 * bs], y[bs:2 * bs], dinv[bs:2 * bs],
      b3, wh1, bh1, wh2, bh2)


def kernel(node_type, edge_type, edge_index, edge_label, node_emb, edge_emb,
           W1, b1, W2, b2, W3, b3, Wh1, bh1, Wh2, bh2):
    del edge_type, edge_emb
    src = edge_index[0].astype(jnp.int32)
    dst = edge_index[1].astype(jnp.int32)
    srcp = jnp.concatenate([src, jnp.zeros((EPAD - E,), jnp.int32)])
    dstp = jnp.concatenate([dst, jnp.full((EPAD - E,), TRASH, jnp.int32)])
    ntp = jnp.pad(node_type.astype(jnp.int32), (0, NP - N))
    zeros2d = jnp.zeros((NP, HID), F32)

    deg2 = _sc_degree(dstp).reshape(2, NP)
    y1, dinv = _tc_prep(deg2, ntp, node_emb, W1)

    z = _sc_scatter(y1, srcp, dstp, zeros2d)
    y2 = _tc_mid(z[:NP], z[NP:], y1, dinv, b1, W2)
    z = _sc_scatter(y2, srcp, dstp, zeros2d)
    y3 = _tc_mid(z[:NP], z[NP:], y2, dinv, b2, W3)
    z = _sc_scatter(y3, srcp, dstp, zeros2d)

    bs = edge_label.shape[0]
    pred = _tc_head(bs, z[:NP], z[NP:], y3, dinv, b3, Wh1, bh1, Wh2, bh2)
    return (pred, edge_label)
